# baseline (speedup 1.0000x reference)
import functools

import jax
import jax.numpy as jnp
from jax.experimental import pallas as pl
from jax.experimental.pallas import tpu as pltpu

N = 10000
E = 160000
H = 256
ED = 16
TD = 128
NB = 4

N_PAD = 10240
NC = 2048
EC = 4000
NSC = 8
W_OUT = 264
TW = 264


def _silu(v):
    return v * jax.nn.sigmoid(v)


def _pq_body(h_ref, x_ref, wr_ref, wc_ref, be1_ref, p_ref, q_ref):
    h = h_ref[...]
    xb = x_ref[...].astype(jnp.bfloat16)
    zero = jnp.zeros((h.shape[0], TW - H - 4), jnp.bfloat16)
    p_ref[:, :H] = (jnp.dot(h, wr_ref[...], preferred_element_type=jnp.float32)
                    + be1_ref[...]).astype(jnp.bfloat16)
    p_ref[:, H:H + 4] = xb
    p_ref[:, H + 4:] = zero
    q_ref[:, :H] = jnp.dot(h, wc_ref[...], preferred_element_type=jnp.float32).astype(jnp.bfloat16)
    q_ref[:, H:H + 4] = xb
    q_ref[:, H + 4:] = zero


def _pq(h_pad, x4, w_row, w_col, be1):
    grid = (N_PAD // NC,)
    bs_n = pl.BlockSpec((NC, H), lambda i: (i, 0))
    bs_x = pl.BlockSpec((NC, 4), lambda i: (i, 0))
    bs_o = pl.BlockSpec((NC, TW), lambda i: (i, 0))
    bs_w = pl.BlockSpec((H, H), lambda i: (0, 0))
    bs_b = pl.BlockSpec((1, H), lambda i: (0, 0))
    return pl.pallas_call(
        _pq_body,
        grid=grid,
        in_specs=[bs_n, bs_x, bs_w, bs_w, bs_b],
        out_specs=[bs_o, bs_o],
        out_shape=[jax.ShapeDtypeStruct((N_PAD, TW), jnp.bfloat16)] * 2,
    )(h_pad, x4, w_row, w_col, be1.reshape(1, H))


def _edge_body(gp_ref, gq_ref, ea_ref,
               wrad_ref, wea_ref, we2_ref, be2_ref, wa_ref, ba_ref,
               wx1_ref, bx1_ref, wx2_ref, bx2_ref,
               out_ref):
    dx = (gp_ref[:, H:H + 4] - gq_ref[:, H:H + 4]).astype(jnp.float32)
    radial = jnp.sum(dx * dx, axis=-1, keepdims=True)
    pre = (gp_ref[:, :H].astype(jnp.float32) + gq_ref[:, :H].astype(jnp.float32)
           + radial * wrad_ref[...]
           + jnp.dot(ea_ref[...], wea_ref[...], preferred_element_type=jnp.float32))
    m1 = _silu(pre).astype(jnp.bfloat16)
    we2 = we2_ref[...].astype(jnp.bfloat16)
    m2 = _silu(jnp.dot(m1, we2, preferred_element_type=jnp.float32) + be2_ref[...])
    att = jax.nn.sigmoid(jnp.dot(m2, wa_ref[...], preferred_element_type=jnp.float32) + ba_ref[0, 0])
    m = m2 * att
    mb = m.astype(jnp.bfloat16)
    wx1 = wx1_ref[...].astype(jnp.bfloat16)
    t = _silu(jnp.dot(mb, wx1, preferred_element_type=jnp.float32) + bx1_ref[...])
    cw = jnp.dot(t, wx2_ref[...], preferred_element_type=jnp.float32) + bx2_ref[0, 0]
    out_ref[:, :H] = m
    out_ref[:, H:H + 4] = dx * cw
    out_ref[:, H + 4:H + 5] = jnp.ones_like(cw)
    out_ref[:, H + 5:] = jnp.zeros((dx.shape[0], 3), jnp.float32)


def _edge_mlp(gp, gq, edge_attr, wrad, wea, we2, be2, wa, ba, wx1, bx1, wx2, bx2):
    ne = gp.shape[0]
    grid = (ne // EC,)
    bs_g = pl.BlockSpec((EC, TW), lambda i: (i, 0))
    bs_o = pl.BlockSpec((EC, W_OUT), lambda i: (i, 0))
    bs_ea = pl.BlockSpec((EC, ED), lambda i: (i, 0))
    cw = lambda s: pl.BlockSpec(s, lambda i: (0, 0))
    return pl.pallas_call(
        _edge_body,
        grid=grid,
        in_specs=[bs_g, bs_g, bs_ea,
                  cw((1, H)), cw((ED, H)), cw((H, H)), cw((1, H)), cw((H, 1)), cw((1, 1)),
                  cw((H, H)), cw((1, H)), cw((H, 1)), cw((1, 1))],
        out_specs=[bs_o],
        out_shape=[jax.ShapeDtypeStruct((ne, W_OUT), jnp.float32)],
    )(gp, gq, edge_attr, wrad, wea, we2, be2, wa, ba, wx1, bx1, wx2, bx2)


def _node_body(h_ref, s_ref, te_ref, x_ref,
               wh_ref, wa_ref, wt_ref, bh1_ref, wh2_ref, bh2_ref,
               ho_ref, xo_ref):
    agg = s_ref[:, :H].astype(jnp.float32)
    xagg = s_ref[:, H:H + 4].astype(jnp.float32)
    cnt = jnp.clip(s_ref[:, H + 4:H + 5].astype(jnp.float32), 1.0, None)
    pre = (jnp.dot(h_ref[...], wh_ref[...], preferred_element_type=jnp.float32)
           + jnp.dot(agg, wa_ref[...], preferred_element_type=jnp.float32)
           + jnp.dot(te_ref[...], wt_ref[...], preferred_element_type=jnp.float32)
           + bh1_ref[...])
    hn = jnp.dot(_silu(pre), wh2_ref[...], preferred_element_type=jnp.float32) + bh2_ref[...]
    ho_ref[...] = h_ref[...] + hn
    xo_ref[...] = x_ref[...] + xagg / cnt


def _node_update(h_pad, S, te_pad, x4, wh, wa, wt, bh1, wh2, bh2):
    grid = (N_PAD // NC,)
    bs_n = pl.BlockSpec((NC, H), lambda i: (i, 0))
    bs_s = pl.BlockSpec((NC, W_OUT), lambda i: (i, 0))
    bs_t = pl.BlockSpec((NC, TD), lambda i: (i, 0))
    bs_x = pl.BlockSpec((NC, 4), lambda i: (i, 0))
    cw = lambda s: pl.BlockSpec(s, lambda i: (0, 0))
    return pl.pallas_call(
        _node_body,
        grid=grid,
        in_specs=[bs_n, bs_s, bs_t, bs_x,
                  cw((H, H)), cw((H, H)), cw((TD, H)), cw((1, H)), cw((H, H)), cw((1, H))],
        out_specs=[bs_n, bs_x],
        out_shape=[jax.ShapeDtypeStruct((N_PAD, H), jnp.float32),
                   jax.ShapeDtypeStruct((N_PAD, 4), jnp.float32)],
    )(h_pad, S, te_pad, x4, wh, wa, wt, bh1.reshape(1, H), wh2, bh2.reshape(1, H))


def kernel(h, x, edge_index, edge_attr, t_emb, We1, be1, We2, be2, Wa, ba,
           Wx1, bx1, Wx2, bx2, Wh1, bh1, Wh2, bh2):
    row = edge_index[0]
    col = edge_index[1]

    h_pad = jnp.zeros((N_PAD, H), jnp.float32).at[:N].set(h)
    x4 = jnp.zeros((N_PAD, 4), jnp.float32).at[:N, :3].set(x)
    te_pad = jnp.zeros((N_PAD, TD), jnp.float32).at[:N].set(t_emb)

    for b in range(NB):
        w_row = We1[b, :H]
        w_col = We1[b, H:2 * H]
        wrad = We1[b, 2 * H:2 * H + 1]
        wea = We1[b, 2 * H + 1:]

        P, Q = _pq(h_pad, x4, w_row, w_col, be1[b])

        S = jnp.zeros((N_PAD, W_OUT), jnp.float32)
        esc = E // NSC
        for k in range(NSC):
            sl = slice(k * esc, (k + 1) * esc)
            gp = jnp.take(P, row[sl], axis=0)
            gq = jnp.take(Q, col[sl], axis=0)
            [out] = _edge_mlp(gp, gq, edge_attr[sl], wrad, wea,
                              We2[b], be2[b].reshape(1, H), Wa[b], ba[b].reshape(1, 1),
                              Wx1[b], bx1[b].reshape(1, H), Wx2[b], bx2[b].reshape(1, 1))
            S = S.at[row[sl]].add(out)

        h_pad, x4 = _node_update(h_pad, S, te_pad, x4,
                                 Wh1[b, :H], Wh1[b, H:2 * H], Wh1[b, 2 * H:],
                                 bh1[b], Wh2[b], bh2[b])

    return h_pad[:N], x4[:N, :3]


# speedup vs baseline: 1.0357x; 1.0357x over previous
import functools

import jax
import jax.numpy as jnp
from jax.experimental import pallas as pl
from jax.experimental.pallas import tpu as pltpu

N = 10000
E = 160000
H = 256
ED = 16
TD = 128
NB = 4

N_PAD = 10240
NC = 2048
EC = 4000
NSC = 4
W_OUT = 264
TW = 264


def _silu(v):
    return v * jax.nn.sigmoid(v)


def _pq_body(h_ref, x_ref, wr_ref, wc_ref, be1_ref, p_ref, q_ref):
    h = h_ref[...]
    xb = x_ref[...].astype(jnp.bfloat16)
    zero = jnp.zeros((h.shape[0], TW - H - 4), jnp.bfloat16)
    p_ref[:, :H] = (jnp.dot(h, wr_ref[...], preferred_element_type=jnp.float32)
                    + be1_ref[...]).astype(jnp.bfloat16)
    p_ref[:, H:H + 4] = xb
    p_ref[:, H + 4:] = zero
    q_ref[:, :H] = jnp.dot(h, wc_ref[...], preferred_element_type=jnp.float32).astype(jnp.bfloat16)
    q_ref[:, H:H + 4] = xb
    q_ref[:, H + 4:] = zero


def _pq(h_pad, x4, w_row, w_col, be1):
    grid = (N_PAD // NC,)
    bs_n = pl.BlockSpec((NC, H), lambda i: (i, 0))
    bs_x = pl.BlockSpec((NC, 4), lambda i: (i, 0))
    bs_o = pl.BlockSpec((NC, TW), lambda i: (i, 0))
    bs_w = pl.BlockSpec((H, H), lambda i: (0, 0))
    bs_b = pl.BlockSpec((1, H), lambda i: (0, 0))
    return pl.pallas_call(
        _pq_body,
        grid=grid,
        in_specs=[bs_n, bs_x, bs_w, bs_w, bs_b],
        out_specs=[bs_o, bs_o],
        out_shape=[jax.ShapeDtypeStruct((N_PAD, TW), jnp.bfloat16)] * 2,
    )(h_pad, x4, w_row, w_col, be1.reshape(1, H))


def _edge_body(gp_ref, gq_ref, ea_ref,
               wrad_ref, wea_ref, we2_ref, be2_ref, wa_ref, ba_ref,
               wx1_ref, bx1_ref, wx2_ref, bx2_ref,
               out_ref):
    dx = (gp_ref[:, H:H + 4] - gq_ref[:, H:H + 4]).astype(jnp.float32)
    radial = jnp.sum(dx * dx, axis=-1, keepdims=True)
    pre = (gp_ref[:, :H].astype(jnp.float32) + gq_ref[:, :H].astype(jnp.float32)
           + radial * wrad_ref[...]
           + jnp.dot(ea_ref[...], wea_ref[...], preferred_element_type=jnp.float32))
    m1 = _silu(pre).astype(jnp.bfloat16)
    we2 = we2_ref[...].astype(jnp.bfloat16)
    m2 = _silu(jnp.dot(m1, we2, preferred_element_type=jnp.float32) + be2_ref[...])
    att = jax.nn.sigmoid(jnp.dot(m2, wa_ref[...], preferred_element_type=jnp.float32) + ba_ref[0, 0])
    m = m2 * att
    mb = m.astype(jnp.bfloat16)
    wx1 = wx1_ref[...].astype(jnp.bfloat16)
    t = _silu(jnp.dot(mb, wx1, preferred_element_type=jnp.float32) + bx1_ref[...])
    cw = jnp.dot(t, wx2_ref[...], preferred_element_type=jnp.float32) + bx2_ref[0, 0]
    out_ref[:, :H] = m
    out_ref[:, H:H + 4] = dx * cw
    out_ref[:, H + 4:H + 5] = jnp.ones_like(cw)
    out_ref[:, H + 5:] = jnp.zeros((dx.shape[0], 3), jnp.float32)


def _edge_mlp(gp, gq, edge_attr, wrad, wea, we2, be2, wa, ba, wx1, bx1, wx2, bx2):
    ne = gp.shape[0]
    grid = (ne // EC,)
    bs_g = pl.BlockSpec((EC, TW), lambda i: (i, 0))
    bs_o = pl.BlockSpec((EC, W_OUT), lambda i: (i, 0))
    bs_ea = pl.BlockSpec((EC, ED), lambda i: (i, 0))
    cw = lambda s: pl.BlockSpec(s, lambda i: (0, 0))
    return pl.pallas_call(
        _edge_body,
        grid=grid,
        in_specs=[bs_g, bs_g, bs_ea,
                  cw((1, H)), cw((ED, H)), cw((H, H)), cw((1, H)), cw((H, 1)), cw((1, 1)),
                  cw((H, H)), cw((1, H)), cw((H, 1)), cw((1, 1))],
        out_specs=[bs_o],
        out_shape=[jax.ShapeDtypeStruct((ne, W_OUT), jnp.float32)],
    )(gp, gq, edge_attr, wrad, wea, we2, be2, wa, ba, wx1, bx1, wx2, bx2)


def _node_body(h_ref, s_ref, te_ref, x_ref,
               wh_ref, wa_ref, wt_ref, bh1_ref, wh2_ref, bh2_ref,
               ho_ref, xo_ref):
    agg = s_ref[:, :H].astype(jnp.float32)
    xagg = s_ref[:, H:H + 4].astype(jnp.float32)
    cnt = jnp.clip(s_ref[:, H + 4:H + 5].astype(jnp.float32), 1.0, None)
    pre = (jnp.dot(h_ref[...], wh_ref[...], preferred_element_type=jnp.float32)
           + jnp.dot(agg, wa_ref[...], preferred_element_type=jnp.float32)
           + jnp.dot(te_ref[...], wt_ref[...], preferred_element_type=jnp.float32)
           + bh1_ref[...])
    hn = jnp.dot(_silu(pre), wh2_ref[...], preferred_element_type=jnp.float32) + bh2_ref[...]
    ho_ref[...] = h_ref[...] + hn
    xo_ref[...] = x_ref[...] + xagg / cnt


def _node_update(h_pad, S, te_pad, x4, wh, wa, wt, bh1, wh2, bh2):
    grid = (N_PAD // NC,)
    bs_n = pl.BlockSpec((NC, H), lambda i: (i, 0))
    bs_s = pl.BlockSpec((NC, W_OUT), lambda i: (i, 0))
    bs_t = pl.BlockSpec((NC, TD), lambda i: (i, 0))
    bs_x = pl.BlockSpec((NC, 4), lambda i: (i, 0))
    cw = lambda s: pl.BlockSpec(s, lambda i: (0, 0))
    return pl.pallas_call(
        _node_body,
        grid=grid,
        in_specs=[bs_n, bs_s, bs_t, bs_x,
                  cw((H, H)), cw((H, H)), cw((TD, H)), cw((1, H)), cw((H, H)), cw((1, H))],
        out_specs=[bs_n, bs_x],
        out_shape=[jax.ShapeDtypeStruct((N_PAD, H), jnp.float32),
                   jax.ShapeDtypeStruct((N_PAD, 4), jnp.float32)],
    )(h_pad, S, te_pad, x4, wh, wa, wt, bh1.reshape(1, H), wh2, bh2.reshape(1, H))


def kernel(h, x, edge_index, edge_attr, t_emb, We1, be1, We2, be2, Wa, ba,
           Wx1, bx1, Wx2, bx2, Wh1, bh1, Wh2, bh2):
    row = edge_index[0]
    col = edge_index[1]

    h_pad = jnp.zeros((N_PAD, H), jnp.float32).at[:N].set(h)
    x4 = jnp.zeros((N_PAD, 4), jnp.float32).at[:N, :3].set(x)
    te_pad = jnp.zeros((N_PAD, TD), jnp.float32).at[:N].set(t_emb)

    for b in range(NB):
        w_row = We1[b, :H]
        w_col = We1[b, H:2 * H]
        wrad = We1[b, 2 * H:2 * H + 1]
        wea = We1[b, 2 * H + 1:]

        P, Q = _pq(h_pad, x4, w_row, w_col, be1[b])

        S = jnp.zeros((N_PAD, W_OUT), jnp.float32)
        esc = E // NSC
        for k in range(NSC):
            sl = slice(k * esc, (k + 1) * esc)
            gp = jnp.take(P, row[sl], axis=0)
            gq = jnp.take(Q, col[sl], axis=0)
            [out] = _edge_mlp(gp, gq, edge_attr[sl], wrad, wea,
                              We2[b], be2[b].reshape(1, H), Wa[b], ba[b].reshape(1, 1),
                              Wx1[b], bx1[b].reshape(1, H), Wx2[b], bx2[b].reshape(1, 1))
            S = S.at[row[sl]].add(out)

        h_pad, x4 = _node_update(h_pad, S, te_pad, x4,
                                 Wh1[b, :H], Wh1[b, H:2 * H], Wh1[b, 2 * H:],
                                 bh1[b], Wh2[b], bh2[b])

    return h_pad[:N], x4[:N, :3]


# speedup vs baseline: 1.1497x; 1.1100x over previous
import functools

import jax
import jax.numpy as jnp
from jax.experimental import pallas as pl
from jax.experimental.pallas import tpu as pltpu

N = 10000
E = 160000
H = 256
ED = 16
TD = 128
NB = 4

N_PAD = 10240
NC = 2048
EC = 4000
NSC = 4
W_OUT = 264
TW = 264


def _silu(v):
    return v * jax.nn.sigmoid(v)


def _pq_body(h_ref, x_ref, wr_ref, wc_ref, be1_ref, p_ref, q_ref):
    h = h_ref[...]
    xb = x_ref[...]
    zero = jnp.zeros((h.shape[0], TW - H - 4), jnp.float32)
    p_ref[:, :H] = (jnp.dot(h, wr_ref[...], preferred_element_type=jnp.float32)
                    + be1_ref[...])
    p_ref[:, H:H + 4] = xb
    p_ref[:, H + 4:] = zero
    q_ref[:, :H] = jnp.dot(h, wc_ref[...], preferred_element_type=jnp.float32)
    q_ref[:, H:H + 4] = -xb
    q_ref[:, H + 4:] = zero


def _pq(h_pad, x4, w_row, w_col, be1):
    grid = (N_PAD // NC,)
    bs_n = pl.BlockSpec((NC, H), lambda i: (i, 0))
    bs_x = pl.BlockSpec((NC, 4), lambda i: (i, 0))
    bs_o = pl.BlockSpec((NC, TW), lambda i: (i, 0))
    bs_w = pl.BlockSpec((H, H), lambda i: (0, 0))
    bs_b = pl.BlockSpec((1, H), lambda i: (0, 0))
    return pl.pallas_call(
        _pq_body,
        grid=grid,
        in_specs=[bs_n, bs_x, bs_w, bs_w, bs_b],
        out_specs=[bs_o, bs_o],
        out_shape=[jax.ShapeDtypeStruct((N_PAD, TW), jnp.float32)] * 2,
    )(h_pad, x4, w_row, w_col, be1.reshape(1, H))


def _edge_body(rows_ref, cols_ref, p_ref, q_ref, ea_ref,
               wrad_ref, wea_ref, we2_ref, be2_ref, wa_ref, ba_ref,
               wx1_ref, bx1_ref, wx2_ref, bx2_ref,
               out_ref, g_ref):
    def gath(e, carry):
        r = rows_ref[0, 0, e]
        c = cols_ref[0, 0, e]
        g_ref[pl.ds(e, 1), :] = p_ref[pl.ds(r, 1), :] + q_ref[pl.ds(c, 1), :]
        return carry

    jax.lax.fori_loop(0, EC, gath, 0, unroll=8)
    g = g_ref[...]
    dx = g[:, H:H + 4]
    radial = jnp.sum(dx * dx, axis=-1, keepdims=True)
    pre = (g[:, :H]
           + radial * wrad_ref[...]
           + jnp.dot(ea_ref[...], wea_ref[...], preferred_element_type=jnp.float32))
    m1 = _silu(pre).astype(jnp.bfloat16)
    we2 = we2_ref[...].astype(jnp.bfloat16)
    m2 = _silu(jnp.dot(m1, we2, preferred_element_type=jnp.float32) + be2_ref[...])
    att = jax.nn.sigmoid(jnp.dot(m2, wa_ref[...], preferred_element_type=jnp.float32) + ba_ref[0, 0])
    m = m2 * att
    mb = m.astype(jnp.bfloat16)
    wx1 = wx1_ref[...].astype(jnp.bfloat16)
    t = _silu(jnp.dot(mb, wx1, preferred_element_type=jnp.float32) + bx1_ref[...])
    cw = jnp.dot(t, wx2_ref[...], preferred_element_type=jnp.float32) + bx2_ref[0, 0]
    out_ref[:, :H] = m
    out_ref[:, H:H + 4] = dx * cw
    out_ref[:, H + 4:H + 5] = jnp.ones_like(cw)
    out_ref[:, H + 5:] = jnp.zeros((dx.shape[0], 3), jnp.float32)


def _edge_mlp(P, Q, rows3, cols3, edge_attr, wrad, wea, we2, be2, wa, ba, wx1, bx1, wx2, bx2):
    ne = rows3.shape[0] * EC
    grid = (ne // EC,)
    bs_i = pl.BlockSpec((1, 1, EC), lambda i: (i, 0, 0), memory_space=pltpu.SMEM)
    bs_t = pl.BlockSpec((N_PAD, TW), lambda i: (0, 0))
    bs_o = pl.BlockSpec((EC, W_OUT), lambda i: (i, 0))
    bs_ea = pl.BlockSpec((EC, ED), lambda i: (i, 0))
    cw = lambda s: pl.BlockSpec(s, lambda i: (0, 0))
    return pl.pallas_call(
        _edge_body,
        grid=grid,
        in_specs=[bs_i, bs_i, bs_t, bs_t, bs_ea,
                  cw((1, H)), cw((ED, H)), cw((H, H)), cw((1, H)), cw((H, 1)), cw((1, 1)),
                  cw((H, H)), cw((1, H)), cw((H, 1)), cw((1, 1))],
        out_specs=[bs_o],
        out_shape=[jax.ShapeDtypeStruct((ne, W_OUT), jnp.float32)],
        scratch_shapes=[pltpu.VMEM((EC, TW), jnp.float32)],
    )(rows3, cols3, P, Q, edge_attr, wrad, wea, we2, be2, wa, ba, wx1, bx1, wx2, bx2)


def _node_body(h_ref, s_ref, te_ref, x_ref,
               wh_ref, wa_ref, wt_ref, bh1_ref, wh2_ref, bh2_ref,
               ho_ref, xo_ref):
    agg = s_ref[:, :H].astype(jnp.float32)
    xagg = s_ref[:, H:H + 4].astype(jnp.float32)
    cnt = jnp.clip(s_ref[:, H + 4:H + 5].astype(jnp.float32), 1.0, None)
    pre = (jnp.dot(h_ref[...], wh_ref[...], preferred_element_type=jnp.float32)
           + jnp.dot(agg, wa_ref[...], preferred_element_type=jnp.float32)
           + jnp.dot(te_ref[...], wt_ref[...], preferred_element_type=jnp.float32)
           + bh1_ref[...])
    hn = jnp.dot(_silu(pre), wh2_ref[...], preferred_element_type=jnp.float32) + bh2_ref[...]
    ho_ref[...] = h_ref[...] + hn
    xo_ref[...] = x_ref[...] + xagg / cnt


def _node_update(h_pad, S, te_pad, x4, wh, wa, wt, bh1, wh2, bh2):
    grid = (N_PAD // NC,)
    bs_n = pl.BlockSpec((NC, H), lambda i: (i, 0))
    bs_s = pl.BlockSpec((NC, W_OUT), lambda i: (i, 0))
    bs_t = pl.BlockSpec((NC, TD), lambda i: (i, 0))
    bs_x = pl.BlockSpec((NC, 4), lambda i: (i, 0))
    cw = lambda s: pl.BlockSpec(s, lambda i: (0, 0))
    return pl.pallas_call(
        _node_body,
        grid=grid,
        in_specs=[bs_n, bs_s, bs_t, bs_x,
                  cw((H, H)), cw((H, H)), cw((TD, H)), cw((1, H)), cw((H, H)), cw((1, H))],
        out_specs=[bs_n, bs_x],
        out_shape=[jax.ShapeDtypeStruct((N_PAD, H), jnp.float32),
                   jax.ShapeDtypeStruct((N_PAD, 4), jnp.float32)],
    )(h_pad, S, te_pad, x4, wh, wa, wt, bh1.reshape(1, H), wh2, bh2.reshape(1, H))


def kernel(h, x, edge_index, edge_attr, t_emb, We1, be1, We2, be2, Wa, ba,
           Wx1, bx1, Wx2, bx2, Wh1, bh1, Wh2, bh2):
    row = edge_index[0]
    col = edge_index[1]

    h_pad = jnp.zeros((N_PAD, H), jnp.float32).at[:N].set(h)
    x4 = jnp.zeros((N_PAD, 4), jnp.float32).at[:N, :3].set(x)
    te_pad = jnp.zeros((N_PAD, TD), jnp.float32).at[:N].set(t_emb)

    for b in range(NB):
        w_row = We1[b, :H]
        w_col = We1[b, H:2 * H]
        wrad = We1[b, 2 * H:2 * H + 1]
        wea = We1[b, 2 * H + 1:]

        P, Q = _pq(h_pad, x4, w_row, w_col, be1[b])

        S = jnp.zeros((N_PAD, W_OUT), jnp.float32)
        esc = E // NSC
        for k in range(NSC):
            sl = slice(k * esc, (k + 1) * esc)
            rows3 = row[sl].reshape(-1, 1, EC)
            cols3 = col[sl].reshape(-1, 1, EC)
            [out] = _edge_mlp(P, Q, rows3, cols3, edge_attr[sl], wrad, wea,
                              We2[b], be2[b].reshape(1, H), Wa[b], ba[b].reshape(1, 1),
                              Wx1[b], bx1[b].reshape(1, H), Wx2[b], bx2[b].reshape(1, 1))
            S = S.at[row[sl]].add(out)

        h_pad, x4 = _node_update(h_pad, S, te_pad, x4,
                                 Wh1[b, :H], Wh1[b, H:2 * H], Wh1[b, 2 * H:],
                                 bh1[b], Wh2[b], bh2[b])

    return h_pad[:N], x4[:N, :3]


# speedup vs baseline: 1.1980x; 1.0420x over previous
import functools

import jax
import jax.numpy as jnp
from jax.experimental import pallas as pl
from jax.experimental.pallas import tpu as pltpu

N = 10000
E = 160000
H = 256
ED = 16
TD = 128
NB = 4

N_PAD = 10240
NC = 2048
EC = 4000
NSC = 4
W_OUT = 264
TW = 264


def _silu(v):
    return v * jax.nn.sigmoid(v)


def _pq_body(h_ref, x_ref, wr_ref, wc_ref, be1_ref, p_ref, q_ref):
    h = h_ref[...]
    xb = x_ref[...]
    zero = jnp.zeros((h.shape[0], TW - H - 4), jnp.float32)
    p_ref[:, :H] = (jnp.dot(h, wr_ref[...], preferred_element_type=jnp.float32)
                    + be1_ref[...])
    p_ref[:, H:H + 4] = xb
    p_ref[:, H + 4:] = zero
    q_ref[:, :H] = jnp.dot(h, wc_ref[...], preferred_element_type=jnp.float32)
    q_ref[:, H:H + 4] = -xb
    q_ref[:, H + 4:] = zero


def _pq(h_pad, x4, w_row, w_col, be1):
    grid = (N_PAD // NC,)
    bs_n = pl.BlockSpec((NC, H), lambda i: (i, 0))
    bs_x = pl.BlockSpec((NC, 4), lambda i: (i, 0))
    bs_o = pl.BlockSpec((NC, TW), lambda i: (i, 0))
    bs_w = pl.BlockSpec((H, H), lambda i: (0, 0))
    bs_b = pl.BlockSpec((1, H), lambda i: (0, 0))
    return pl.pallas_call(
        _pq_body,
        grid=grid,
        in_specs=[bs_n, bs_x, bs_w, bs_w, bs_b],
        out_specs=[bs_o, bs_o],
        out_shape=[jax.ShapeDtypeStruct((N_PAD, TW), jnp.float32)] * 2,
    )(h_pad, x4, w_row, w_col, be1.reshape(1, H))


def _edge_body(rows_ref, cols_ref, p_ref, q_ref, ea_ref,
               wrad_ref, wea_ref, we2_ref, be2_ref, wa_ref, ba_ref,
               wx1_ref, bx1_ref, wx2_ref, bx2_ref,
               out_ref, g_ref):
    def gath(e, carry):
        r = rows_ref[0, 0, e]
        c = cols_ref[0, 0, e]
        g_ref[pl.ds(e, 1), :] = p_ref[pl.ds(r, 1), :] + q_ref[pl.ds(c, 1), :]
        return carry

    jax.lax.fori_loop(0, EC, gath, 0, unroll=16)
    g = g_ref[...]
    dx = g[:, H:H + 4]
    radial = jnp.sum(dx * dx, axis=-1, keepdims=True)
    pre = (g[:, :H]
           + radial * wrad_ref[...]
           + jnp.dot(ea_ref[...], wea_ref[...], preferred_element_type=jnp.float32))
    m1 = _silu(pre).astype(jnp.bfloat16)
    we2 = we2_ref[...].astype(jnp.bfloat16)
    m2 = _silu(jnp.dot(m1, we2, preferred_element_type=jnp.float32) + be2_ref[...])
    att = jax.nn.sigmoid(jnp.dot(m2, wa_ref[...], preferred_element_type=jnp.float32) + ba_ref[0, 0])
    m = m2 * att
    mb = m.astype(jnp.bfloat16)
    wx1 = wx1_ref[...].astype(jnp.bfloat16)
    t = _silu(jnp.dot(mb, wx1, preferred_element_type=jnp.float32) + bx1_ref[...])
    cw = jnp.dot(t, wx2_ref[...], preferred_element_type=jnp.float32) + bx2_ref[0, 0]
    out_ref[:, :H] = m
    out_ref[:, H:H + 4] = dx * cw
    out_ref[:, H + 4:H + 5] = jnp.ones_like(cw)
    out_ref[:, H + 5:] = jnp.zeros((dx.shape[0], 3), jnp.float32)


def _edge_mlp(P, Q, rows3, cols3, edge_attr, wrad, wea, we2, be2, wa, ba, wx1, bx1, wx2, bx2):
    ne = rows3.shape[0] * EC
    grid = (ne // EC,)
    bs_i = pl.BlockSpec((1, 1, EC), lambda i: (i, 0, 0), memory_space=pltpu.SMEM)
    bs_t = pl.BlockSpec((N_PAD, TW), lambda i: (0, 0))
    bs_o = pl.BlockSpec((EC, W_OUT), lambda i: (i, 0))
    bs_ea = pl.BlockSpec((EC, ED), lambda i: (i, 0))
    cw = lambda s: pl.BlockSpec(s, lambda i: (0, 0))
    return pl.pallas_call(
        _edge_body,
        grid=grid,
        in_specs=[bs_i, bs_i, bs_t, bs_t, bs_ea,
                  cw((1, H)), cw((ED, H)), cw((H, H)), cw((1, H)), cw((H, 1)), cw((1, 1)),
                  cw((H, H)), cw((1, H)), cw((H, 1)), cw((1, 1))],
        out_specs=[bs_o],
        out_shape=[jax.ShapeDtypeStruct((ne, W_OUT), jnp.float32)],
        scratch_shapes=[pltpu.VMEM((EC, TW), jnp.float32)],
    )(rows3, cols3, P, Q, edge_attr, wrad, wea, we2, be2, wa, ba, wx1, bx1, wx2, bx2)


def _node_body(h_ref, s_ref, te_ref, x_ref,
               wh_ref, wa_ref, wt_ref, bh1_ref, wh2_ref, bh2_ref,
               ho_ref, xo_ref):
    agg = s_ref[:, :H].astype(jnp.float32)
    xagg = s_ref[:, H:H + 4].astype(jnp.float32)
    cnt = jnp.clip(s_ref[:, H + 4:H + 5].astype(jnp.float32), 1.0, None)
    pre = (jnp.dot(h_ref[...], wh_ref[...], preferred_element_type=jnp.float32)
           + jnp.dot(agg, wa_ref[...], preferred_element_type=jnp.float32)
           + jnp.dot(te_ref[...], wt_ref[...], preferred_element_type=jnp.float32)
           + bh1_ref[...])
    hn = jnp.dot(_silu(pre), wh2_ref[...], preferred_element_type=jnp.float32) + bh2_ref[...]
    ho_ref[...] = h_ref[...] + hn
    xo_ref[...] = x_ref[...] + xagg / cnt


def _node_update(h_pad, S, te_pad, x4, wh, wa, wt, bh1, wh2, bh2):
    grid = (N_PAD // NC,)
    bs_n = pl.BlockSpec((NC, H), lambda i: (i, 0))
    bs_s = pl.BlockSpec((NC, W_OUT), lambda i: (i, 0))
    bs_t = pl.BlockSpec((NC, TD), lambda i: (i, 0))
    bs_x = pl.BlockSpec((NC, 4), lambda i: (i, 0))
    cw = lambda s: pl.BlockSpec(s, lambda i: (0, 0))
    return pl.pallas_call(
        _node_body,
        grid=grid,
        in_specs=[bs_n, bs_s, bs_t, bs_x,
                  cw((H, H)), cw((H, H)), cw((TD, H)), cw((1, H)), cw((H, H)), cw((1, H))],
        out_specs=[bs_n, bs_x],
        out_shape=[jax.ShapeDtypeStruct((N_PAD, H), jnp.float32),
                   jax.ShapeDtypeStruct((N_PAD, 4), jnp.float32)],
    )(h_pad, S, te_pad, x4, wh, wa, wt, bh1.reshape(1, H), wh2, bh2.reshape(1, H))


def kernel(h, x, edge_index, edge_attr, t_emb, We1, be1, We2, be2, Wa, ba,
           Wx1, bx1, Wx2, bx2, Wh1, bh1, Wh2, bh2):
    row = edge_index[0]
    col = edge_index[1]

    h_pad = jnp.zeros((N_PAD, H), jnp.float32).at[:N].set(h)
    x4 = jnp.zeros((N_PAD, 4), jnp.float32).at[:N, :3].set(x)
    te_pad = jnp.zeros((N_PAD, TD), jnp.float32).at[:N].set(t_emb)

    for b in range(NB):
        w_row = We1[b, :H]
        w_col = We1[b, H:2 * H]
        wrad = We1[b, 2 * H:2 * H + 1]
        wea = We1[b, 2 * H + 1:]

        P, Q = _pq(h_pad, x4, w_row, w_col, be1[b])

        S = jnp.zeros((N_PAD, W_OUT), jnp.float32)
        esc = E // NSC
        for k in range(NSC):
            sl = slice(k * esc, (k + 1) * esc)
            rows3 = row[sl].reshape(-1, 1, EC)
            cols3 = col[sl].reshape(-1, 1, EC)
            [out] = _edge_mlp(P, Q, rows3, cols3, edge_attr[sl], wrad, wea,
                              We2[b], be2[b].reshape(1, H), Wa[b], ba[b].reshape(1, 1),
                              Wx1[b], bx1[b].reshape(1, H), Wx2[b], bx2[b].reshape(1, 1))
            S = S.at[row[sl]].add(out)

        h_pad, x4 = _node_update(h_pad, S, te_pad, x4,
                                 Wh1[b, :H], Wh1[b, H:2 * H], Wh1[b, 2 * H:],
                                 bh1[b], Wh2[b], bh2[b])

    return h_pad[:N], x4[:N, :3]


# speedup vs baseline: 1.2239x; 1.0216x over previous
import functools

import jax
import jax.numpy as jnp
from jax.experimental import pallas as pl
from jax.experimental.pallas import tpu as pltpu

N = 10000
E = 160000
H = 256
ED = 16
TD = 128
NB = 4

N_PAD = 10240
NC = 2048
EC = 4000
NSC = 4
W_OUT = 264
TW = 264


def _silu(v):
    return v * jax.nn.sigmoid(v)


def _pq_body(h_ref, x_ref, wr_ref, wc_ref, be1_ref, p_ref, q_ref):
    h = h_ref[...]
    xb = x_ref[...]
    zero = jnp.zeros((h.shape[0], TW - H - 4), jnp.float32)
    p_ref[:, :H] = (jnp.dot(h, wr_ref[...], preferred_element_type=jnp.float32)
                    + be1_ref[...])
    p_ref[:, H:H + 4] = xb
    p_ref[:, H + 4:] = zero
    q_ref[:, :H] = jnp.dot(h, wc_ref[...], preferred_element_type=jnp.float32)
    q_ref[:, H:H + 4] = -xb
    q_ref[:, H + 4:] = zero


def _pq(h_pad, x4, w_row, w_col, be1):
    grid = (N_PAD // NC,)
    bs_n = pl.BlockSpec((NC, H), lambda i: (i, 0))
    bs_x = pl.BlockSpec((NC, 4), lambda i: (i, 0))
    bs_o = pl.BlockSpec((NC, TW), lambda i: (i, 0))
    bs_w = pl.BlockSpec((H, H), lambda i: (0, 0))
    bs_b = pl.BlockSpec((1, H), lambda i: (0, 0))
    return pl.pallas_call(
        _pq_body,
        grid=grid,
        in_specs=[bs_n, bs_x, bs_w, bs_w, bs_b],
        out_specs=[bs_o, bs_o],
        out_shape=[jax.ShapeDtypeStruct((N_PAD, TW), jnp.float32)] * 2,
    )(h_pad, x4, w_row, w_col, be1.reshape(1, H))


def _edge_body(rows_ref, cols_ref, p_ref, q_ref, ea_ref,
               wrad_ref, wea_ref, we2_ref, be2_ref, wa_ref, ba_ref,
               wx1_ref, bx1_ref, wx2_ref, bx2_ref,
               out_ref, g_ref):
    def gath(e, carry):
        r = rows_ref[0, 0, e]
        c = cols_ref[0, 0, e]
        g_ref[pl.ds(e, 1), :] = p_ref[pl.ds(r, 1), :] + q_ref[pl.ds(c, 1), :]
        return carry

    jax.lax.fori_loop(0, EC, gath, 0, unroll=32)
    g = g_ref[...]
    dx = g[:, H:H + 4]
    radial = jnp.sum(dx * dx, axis=-1, keepdims=True)
    pre = (g[:, :H]
           + radial * wrad_ref[...]
           + jnp.dot(ea_ref[...], wea_ref[...], preferred_element_type=jnp.float32))
    m1 = _silu(pre).astype(jnp.bfloat16)
    we2 = we2_ref[...].astype(jnp.bfloat16)
    m2 = _silu(jnp.dot(m1, we2, preferred_element_type=jnp.float32) + be2_ref[...])
    att = jax.nn.sigmoid(jnp.dot(m2, wa_ref[...], preferred_element_type=jnp.float32) + ba_ref[0, 0])
    m = m2 * att
    mb = m.astype(jnp.bfloat16)
    wx1 = wx1_ref[...].astype(jnp.bfloat16)
    t = _silu(jnp.dot(mb, wx1, preferred_element_type=jnp.float32) + bx1_ref[...])
    cw = jnp.dot(t, wx2_ref[...], preferred_element_type=jnp.float32) + bx2_ref[0, 0]
    out_ref[:, :H] = m
    out_ref[:, H:H + 4] = dx * cw
    out_ref[:, H + 4:H + 5] = jnp.ones_like(cw)
    out_ref[:, H + 5:] = jnp.zeros((dx.shape[0], 3), jnp.float32)


def _edge_mlp(P, Q, rows3, cols3, edge_attr, wrad, wea, we2, be2, wa, ba, wx1, bx1, wx2, bx2):
    ne = rows3.shape[0] * EC
    grid = (ne // EC,)
    bs_i = pl.BlockSpec((1, 1, EC), lambda i: (i, 0, 0), memory_space=pltpu.SMEM)
    bs_t = pl.BlockSpec((N_PAD, TW), lambda i: (0, 0))
    bs_o = pl.BlockSpec((EC, W_OUT), lambda i: (i, 0))
    bs_ea = pl.BlockSpec((EC, ED), lambda i: (i, 0))
    cw = lambda s: pl.BlockSpec(s, lambda i: (0, 0))
    return pl.pallas_call(
        _edge_body,
        grid=grid,
        in_specs=[bs_i, bs_i, bs_t, bs_t, bs_ea,
                  cw((1, H)), cw((ED, H)), cw((H, H)), cw((1, H)), cw((H, 1)), cw((1, 1)),
                  cw((H, H)), cw((1, H)), cw((H, 1)), cw((1, 1))],
        out_specs=[bs_o],
        out_shape=[jax.ShapeDtypeStruct((ne, W_OUT), jnp.float32)],
        scratch_shapes=[pltpu.VMEM((EC, TW), jnp.float32)],
    )(rows3, cols3, P, Q, edge_attr, wrad, wea, we2, be2, wa, ba, wx1, bx1, wx2, bx2)


def _node_body(h_ref, s_ref, te_ref, x_ref,
               wh_ref, wa_ref, wt_ref, bh1_ref, wh2_ref, bh2_ref,
               ho_ref, xo_ref):
    agg = s_ref[:, :H].astype(jnp.float32)
    xagg = s_ref[:, H:H + 4].astype(jnp.float32)
    cnt = jnp.clip(s_ref[:, H + 4:H + 5].astype(jnp.float32), 1.0, None)
    pre = (jnp.dot(h_ref[...], wh_ref[...], preferred_element_type=jnp.float32)
           + jnp.dot(agg, wa_ref[...], preferred_element_type=jnp.float32)
           + jnp.dot(te_ref[...], wt_ref[...], preferred_element_type=jnp.float32)
           + bh1_ref[...])
    hn = jnp.dot(_silu(pre), wh2_ref[...], preferred_element_type=jnp.float32) + bh2_ref[...]
    ho_ref[...] = h_ref[...] + hn
    xo_ref[...] = x_ref[...] + xagg / cnt


def _node_update(h_pad, S, te_pad, x4, wh, wa, wt, bh1, wh2, bh2):
    grid = (N_PAD // NC,)
    bs_n = pl.BlockSpec((NC, H), lambda i: (i, 0))
    bs_s = pl.BlockSpec((NC, W_OUT), lambda i: (i, 0))
    bs_t = pl.BlockSpec((NC, TD), lambda i: (i, 0))
    bs_x = pl.BlockSpec((NC, 4), lambda i: (i, 0))
    cw = lambda s: pl.BlockSpec(s, lambda i: (0, 0))
    return pl.pallas_call(
        _node_body,
        grid=grid,
        in_specs=[bs_n, bs_s, bs_t, bs_x,
                  cw((H, H)), cw((H, H)), cw((TD, H)), cw((1, H)), cw((H, H)), cw((1, H))],
        out_specs=[bs_n, bs_x],
        out_shape=[jax.ShapeDtypeStruct((N_PAD, H), jnp.float32),
                   jax.ShapeDtypeStruct((N_PAD, 4), jnp.float32)],
    )(h_pad, S, te_pad, x4, wh, wa, wt, bh1.reshape(1, H), wh2, bh2.reshape(1, H))


def kernel(h, x, edge_index, edge_attr, t_emb, We1, be1, We2, be2, Wa, ba,
           Wx1, bx1, Wx2, bx2, Wh1, bh1, Wh2, bh2):
    row = edge_index[0]
    col = edge_index[1]

    h_pad = jnp.zeros((N_PAD, H), jnp.float32).at[:N].set(h)
    x4 = jnp.zeros((N_PAD, 4), jnp.float32).at[:N, :3].set(x)
    te_pad = jnp.zeros((N_PAD, TD), jnp.float32).at[:N].set(t_emb)

    for b in range(NB):
        w_row = We1[b, :H]
        w_col = We1[b, H:2 * H]
        wrad = We1[b, 2 * H:2 * H + 1]
        wea = We1[b, 2 * H + 1:]

        P, Q = _pq(h_pad, x4, w_row, w_col, be1[b])

        S = jnp.zeros((N_PAD, W_OUT), jnp.float32)
        esc = E // NSC
        for k in range(NSC):
            sl = slice(k * esc, (k + 1) * esc)
            rows3 = row[sl].reshape(-1, 1, EC)
            cols3 = col[sl].reshape(-1, 1, EC)
            [out] = _edge_mlp(P, Q, rows3, cols3, edge_attr[sl], wrad, wea,
                              We2[b], be2[b].reshape(1, H), Wa[b], ba[b].reshape(1, 1),
                              Wx1[b], bx1[b].reshape(1, H), Wx2[b], bx2[b].reshape(1, 1))
            S = S.at[row[sl]].add(out)

        h_pad, x4 = _node_update(h_pad, S, te_pad, x4,
                                 Wh1[b, :H], Wh1[b, H:2 * H], Wh1[b, 2 * H:],
                                 bh1[b], Wh2[b], bh2[b])

    return h_pad[:N], x4[:N, :3]


# speedup vs baseline: 1.2420x; 1.0148x over previous
import functools

import jax
import jax.numpy as jnp
from jax.experimental import pallas as pl
from jax.experimental.pallas import tpu as pltpu

N = 10000
E = 160000
H = 256
ED = 16
TD = 128
NB = 4

N_PAD = 10240
NC = 2048
EC = 4000
NSC = 4
W_OUT = 264
TW = 264


def _silu(v):
    return v * jax.nn.sigmoid(v)


def _pq_body(h_ref, x_ref, wr_ref, wc_ref, be1_ref, p_ref, q_ref):
    h = h_ref[...]
    xb = x_ref[...]
    zero = jnp.zeros((h.shape[0], TW - H - 4), jnp.float32)
    p_ref[:, :H] = (jnp.dot(h, wr_ref[...], preferred_element_type=jnp.float32)
                    + be1_ref[...])
    p_ref[:, H:H + 4] = xb
    p_ref[:, H + 4:] = zero
    q_ref[:, :H] = jnp.dot(h, wc_ref[...], preferred_element_type=jnp.float32)
    q_ref[:, H:H + 4] = -xb
    q_ref[:, H + 4:] = zero


def _pq(h_pad, x4, w_row, w_col, be1):
    grid = (N_PAD // NC,)
    bs_n = pl.BlockSpec((NC, H), lambda i: (i, 0))
    bs_x = pl.BlockSpec((NC, 4), lambda i: (i, 0))
    bs_o = pl.BlockSpec((NC, TW), lambda i: (i, 0))
    bs_w = pl.BlockSpec((H, H), lambda i: (0, 0))
    bs_b = pl.BlockSpec((1, H), lambda i: (0, 0))
    return pl.pallas_call(
        _pq_body,
        grid=grid,
        in_specs=[bs_n, bs_x, bs_w, bs_w, bs_b],
        out_specs=[bs_o, bs_o],
        out_shape=[jax.ShapeDtypeStruct((N_PAD, TW), jnp.float32)] * 2,
    )(h_pad, x4, w_row, w_col, be1.reshape(1, H))


def _edge_body(rows_ref, cols_ref, p_ref, q_ref, ea_ref,
               wrad_ref, wea_ref, we2_ref, be2_ref, wa_ref, ba_ref,
               wx1_ref, bx1_ref, wx2_ref, bx2_ref,
               out_ref, g_ref):
    def gath(e, carry):
        r = rows_ref[0, 0, e]
        c = cols_ref[0, 0, e]
        g_ref[pl.ds(e, 1), :] = p_ref[pl.ds(r, 1), :] + q_ref[pl.ds(c, 1), :]
        return carry

    jax.lax.fori_loop(0, EC, gath, 0, unroll=64)
    g = g_ref[...]
    dx = g[:, H:H + 4]
    radial = jnp.sum(dx * dx, axis=-1, keepdims=True)
    pre = (g[:, :H]
           + radial * wrad_ref[...]
           + jnp.dot(ea_ref[...], wea_ref[...], preferred_element_type=jnp.float32))
    m1 = _silu(pre).astype(jnp.bfloat16)
    we2 = we2_ref[...].astype(jnp.bfloat16)
    m2 = _silu(jnp.dot(m1, we2, preferred_element_type=jnp.float32) + be2_ref[...])
    att = jax.nn.sigmoid(jnp.dot(m2, wa_ref[...], preferred_element_type=jnp.float32) + ba_ref[0, 0])
    m = m2 * att
    mb = m.astype(jnp.bfloat16)
    wx1 = wx1_ref[...].astype(jnp.bfloat16)
    t = _silu(jnp.dot(mb, wx1, preferred_element_type=jnp.float32) + bx1_ref[...])
    cw = jnp.dot(t, wx2_ref[...], preferred_element_type=jnp.float32) + bx2_ref[0, 0]
    out_ref[:, :H] = m
    out_ref[:, H:H + 4] = dx * cw
    out_ref[:, H + 4:H + 5] = jnp.ones_like(cw)
    out_ref[:, H + 5:] = jnp.zeros((dx.shape[0], 3), jnp.float32)


def _edge_mlp(P, Q, rows3, cols3, edge_attr, wrad, wea, we2, be2, wa, ba, wx1, bx1, wx2, bx2):
    ne = rows3.shape[0] * EC
    grid = (ne // EC,)
    bs_i = pl.BlockSpec((1, 1, EC), lambda i: (i, 0, 0), memory_space=pltpu.SMEM)
    bs_t = pl.BlockSpec((N_PAD, TW), lambda i: (0, 0))
    bs_o = pl.BlockSpec((EC, W_OUT), lambda i: (i, 0))
    bs_ea = pl.BlockSpec((EC, ED), lambda i: (i, 0))
    cw = lambda s: pl.BlockSpec(s, lambda i: (0, 0))
    return pl.pallas_call(
        _edge_body,
        grid=grid,
        in_specs=[bs_i, bs_i, bs_t, bs_t, bs_ea,
                  cw((1, H)), cw((ED, H)), cw((H, H)), cw((1, H)), cw((H, 1)), cw((1, 1)),
                  cw((H, H)), cw((1, H)), cw((H, 1)), cw((1, 1))],
        out_specs=[bs_o],
        out_shape=[jax.ShapeDtypeStruct((ne, W_OUT), jnp.float32)],
        scratch_shapes=[pltpu.VMEM((EC, TW), jnp.float32)],
    )(rows3, cols3, P, Q, edge_attr, wrad, wea, we2, be2, wa, ba, wx1, bx1, wx2, bx2)


def _node_body(h_ref, s_ref, te_ref, x_ref,
               wh_ref, wa_ref, wt_ref, bh1_ref, wh2_ref, bh2_ref,
               ho_ref, xo_ref):
    agg = s_ref[:, :H].astype(jnp.float32)
    xagg = s_ref[:, H:H + 4].astype(jnp.float32)
    cnt = jnp.clip(s_ref[:, H + 4:H + 5].astype(jnp.float32), 1.0, None)
    pre = (jnp.dot(h_ref[...], wh_ref[...], preferred_element_type=jnp.float32)
           + jnp.dot(agg, wa_ref[...], preferred_element_type=jnp.float32)
           + jnp.dot(te_ref[...], wt_ref[...], preferred_element_type=jnp.float32)
           + bh1_ref[...])
    hn = jnp.dot(_silu(pre), wh2_ref[...], preferred_element_type=jnp.float32) + bh2_ref[...]
    ho_ref[...] = h_ref[...] + hn
    xo_ref[...] = x_ref[...] + xagg / cnt


def _node_update(h_pad, S, te_pad, x4, wh, wa, wt, bh1, wh2, bh2):
    grid = (N_PAD // NC,)
    bs_n = pl.BlockSpec((NC, H), lambda i: (i, 0))
    bs_s = pl.BlockSpec((NC, W_OUT), lambda i: (i, 0))
    bs_t = pl.BlockSpec((NC, TD), lambda i: (i, 0))
    bs_x = pl.BlockSpec((NC, 4), lambda i: (i, 0))
    cw = lambda s: pl.BlockSpec(s, lambda i: (0, 0))
    return pl.pallas_call(
        _node_body,
        grid=grid,
        in_specs=[bs_n, bs_s, bs_t, bs_x,
                  cw((H, H)), cw((H, H)), cw((TD, H)), cw((1, H)), cw((H, H)), cw((1, H))],
        out_specs=[bs_n, bs_x],
        out_shape=[jax.ShapeDtypeStruct((N_PAD, H), jnp.float32),
                   jax.ShapeDtypeStruct((N_PAD, 4), jnp.float32)],
    )(h_pad, S, te_pad, x4, wh, wa, wt, bh1.reshape(1, H), wh2, bh2.reshape(1, H))


def kernel(h, x, edge_index, edge_attr, t_emb, We1, be1, We2, be2, Wa, ba,
           Wx1, bx1, Wx2, bx2, Wh1, bh1, Wh2, bh2):
    row = edge_index[0]
    col = edge_index[1]

    h_pad = jnp.zeros((N_PAD, H), jnp.float32).at[:N].set(h)
    x4 = jnp.zeros((N_PAD, 4), jnp.float32).at[:N, :3].set(x)
    te_pad = jnp.zeros((N_PAD, TD), jnp.float32).at[:N].set(t_emb)

    for b in range(NB):
        w_row = We1[b, :H]
        w_col = We1[b, H:2 * H]
        wrad = We1[b, 2 * H:2 * H + 1]
        wea = We1[b, 2 * H + 1:]

        P, Q = _pq(h_pad, x4, w_row, w_col, be1[b])

        S = jnp.zeros((N_PAD, W_OUT), jnp.float32)
        esc = E // NSC
        for k in range(NSC):
            sl = slice(k * esc, (k + 1) * esc)
            rows3 = row[sl].reshape(-1, 1, EC)
            cols3 = col[sl].reshape(-1, 1, EC)
            [out] = _edge_mlp(P, Q, rows3, cols3, edge_attr[sl], wrad, wea,
                              We2[b], be2[b].reshape(1, H), Wa[b], ba[b].reshape(1, 1),
                              Wx1[b], bx1[b].reshape(1, H), Wx2[b], bx2[b].reshape(1, 1))
            S = S.at[row[sl]].add(out)

        h_pad, x4 = _node_update(h_pad, S, te_pad, x4,
                                 Wh1[b, :H], Wh1[b, H:2 * H], Wh1[b, 2 * H:],
                                 bh1[b], Wh2[b], bh2[b])

    return h_pad[:N], x4[:N, :3]
